# Initial kernel scaffold; baseline (speedup 1.0000x reference)
import jax
import jax.numpy as jnp
from jax.experimental import pallas as pl


def kernel(x, edge_index, edge_attr, W, b):
    raise NotImplementedError("write your pallas kernel here")


# speedup vs baseline: 4.5509x; 4.5509x over previous
import functools

import jax
import jax.numpy as jnp
from jax import lax
from jax.experimental import pallas as pl
from jax.experimental.pallas import tpu as pltpu
from jax.experimental.pallas import tpu_sc as plsc

N = 10000
C = 128
H = 4
NP = 10240
NPW = NP * 4
NC = 2
NS = 16
WORKERS = NC * NS
CHUNK = 128
GRPS = CHUNK // 16
RED = NPW // 16
NEG = -3.4e38


def _iota16():
    return lax.iota(jnp.int32, 16)


def _take(v, idx):
    return jnp.take_along_axis(v, idx, axis=0, mode="promise_in_bounds")


def _tc_tables_body(x_ref, w_ref, b_ref, tab_ref):
    t = jnp.dot(x_ref[...], w_ref[...], preferred_element_type=jnp.float32)
    t = t + b_ref[...]
    tab_ref[0:N, :] = t
    tab_ref[N:NP, :] = jnp.zeros((NP - N, 8), jnp.float32)


def _alpha16(rrows, crows, pos, ev):
    aev = jnp.abs(ev)
    out = []
    for k in range(H):
        ar = plsc.load_gather(rrows, [pos, jnp.full((16,), k, jnp.int32)])
        ac = plsc.load_gather(crows, [pos, jnp.full((16,), 4 + k, jnp.int32)])
        s = (ar + ac) * aev
        out.append(jnp.where(s >= 0.0, s, 0.2 * s) * 100.0)
    return out


def _sc_amax_body(ew, nchunk, row_h, col_h, ea_h, tab_h, parts_h,
                  amax_v, rbuf, cbuf, ebuf, rrows, crows):
    wid = lax.axis_index("s") * NC + lax.axis_index("c")

    def init_i(i, _):
        amax_v[pl.ds(i * 16, 16)] = jnp.full((16,), NEG, jnp.float32)
        return 0
    lax.fori_loop(0, NPW // 16, init_i, 0)

    iota = _iota16()
    ebase = wid * ew

    def chunk_i(i, _):
        base = ebase + i * CHUNK
        pltpu.sync_copy(row_h.at[pl.ds(base, CHUNK)], rbuf)
        pltpu.sync_copy(col_h.at[pl.ds(base, CHUNK)], cbuf)
        pltpu.sync_copy(ea_h.at[pl.ds(base, CHUNK)], ebuf)
        pltpu.sync_copy(tab_h.at[rbuf], rrows)
        pltpu.sync_copy(tab_h.at[cbuf], crows)

        def grp_g(g, _):
            sl = pl.ds(g * 16, 16)
            rs, perm = plsc.sort_key_val(rbuf[sl], iota)
            pos = perm + g * 16
            es = _take(ebuf[sl], perm)
            al = _alpha16(rrows, crows, pos, es)
            for d in (1, 2, 4, 8):
                idxd = jnp.maximum(iota - d, 0)
                eq = _take(rs, idxd) == rs
                for k in range(H):
                    vsh = _take(al[k], idxd)
                    al[k] = jnp.where(eq, jnp.maximum(al[k], vsh), al[k])
            nxt = _take(rs, jnp.minimum(iota + 1, 15))
            lastm = (rs != nxt) | (iota == 15)
            r4 = rs * 4
            for k in range(H):
                cur = plsc.load_gather(amax_v, [r4 + k], mask=lastm)
                plsc.store_scatter(amax_v, [r4 + k],
                                   jnp.maximum(cur, al[k]), mask=lastm)
            return 0
        lax.fori_loop(0, GRPS, grp_g, 0)
        return 0
    lax.fori_loop(0, nchunk, chunk_i, 0)
    pltpu.sync_copy(amax_v, parts_h.at[pl.ds(wid * NPW, NPW)])


def _sc_reduce_body(parts_h, amaxg_h, acc, tmp):
    wid = lax.axis_index("s") * NC + lax.axis_index("c")
    seg = wid * RED

    @pl.when(wid < 16)
    def _():
        pltpu.sync_copy(parts_h.at[pl.ds(seg, RED)], acc)

        def tab_t(t, _):
            pltpu.sync_copy(parts_h.at[pl.ds(t * NPW + seg, RED)], tmp)

            def red_i(i, _):
                sl = pl.ds(i * 16, 16)
                acc[sl] = jnp.maximum(acc[sl], tmp[sl])
                return 0
            lax.fori_loop(0, RED // 16, red_i, 0)
            return 0
        lax.fori_loop(1, WORKERS, tab_t, 0)
        pltpu.sync_copy(acc, amaxg_h.at[pl.ds(seg, RED)])


def _sc_ex_body(ew, nchunk, row_h, col_h, ea_h, tab_h, amaxg_h, zer_h,
                ex_h, sparts_h, rbuf, cbuf, ebuf, rrows, crows, mrows,
                exstage, ssum_sh):
    c = lax.axis_index("c")
    s = lax.axis_index("s")
    wid = s * NC + c
    rows_per = NP // NS
    pltpu.sync_copy(zer_h.at[pl.ds(s * rows_per, rows_per)],
                    ssum_sh.at[pl.ds(s * rows_per, rows_per)])
    plsc.subcore_barrier()

    iota = _iota16()
    ebase = wid * ew

    def chunk_i(i, _):
        base = ebase + i * CHUNK
        pltpu.sync_copy(row_h.at[pl.ds(base, CHUNK)], rbuf)
        pltpu.sync_copy(col_h.at[pl.ds(base, CHUNK)], cbuf)
        pltpu.sync_copy(ea_h.at[pl.ds(base, CHUNK)], ebuf)
        pltpu.sync_copy(tab_h.at[rbuf], rrows)
        pltpu.sync_copy(tab_h.at[cbuf], crows)
        pltpu.sync_copy(amaxg_h.at[rbuf], mrows)

        def grp_g(g, _):
            sl = pl.ds(g * 16, 16)
            pos = iota + g * 16
            al = _alpha16(rrows, crows, pos, ebuf[sl])
            for k in range(H):
                kk = jnp.full((16,), k, jnp.int32)
                m = plsc.load_gather(mrows, [pos, kk])
                plsc.store_scatter(exstage, [pos, kk], jnp.exp(al[k] - m))
            return 0
        lax.fori_loop(0, GRPS, grp_g, 0)
        pltpu.sync_copy(exstage, ex_h.at[pl.ds(base, CHUNK)])
        pltpu.sync_copy(exstage, ssum_sh.at[rbuf], add=True)
        return 0
    lax.fori_loop(0, nchunk, chunk_i, 0)
    plsc.subcore_barrier()

    @pl.when(s == 0)
    def _():
        pltpu.sync_copy(ssum_sh, sparts_h.at[c])


def _sc_norm_body(ew, nchunk, row_h, ex_h, sparts_h, out_h,
                  rbuf, exch, s0rows, s1rows, outst):
    wid = lax.axis_index("s") * NC + lax.axis_index("c")
    iota = _iota16()
    ebase = wid * ew

    def chunk_i(i, _):
        base = ebase + i * CHUNK
        pltpu.sync_copy(row_h.at[pl.ds(base, CHUNK)], rbuf)
        pltpu.sync_copy(ex_h.at[pl.ds(base, CHUNK)], exch)
        pltpu.sync_copy(sparts_h.at[0].at[rbuf], s0rows)
        pltpu.sync_copy(sparts_h.at[1].at[rbuf], s1rows)

        def grp_g(g, _):
            pos = iota + g * 16
            for k in range(H):
                kk = jnp.full((16,), k, jnp.int32)
                ex = plsc.load_gather(exch, [pos, kk])
                u0 = plsc.load_gather(s0rows, [pos, kk])
                u1 = plsc.load_gather(s1rows, [pos, kk])
                plsc.store_scatter(outst, [pos, kk], ex / (u0 + u1 + 1e-16))
            return 0
        lax.fori_loop(0, GRPS, grp_g, 0)
        pltpu.sync_copy(outst, out_h.at[pl.ds(base, CHUNK)])
        return 0
    lax.fori_loop(0, nchunk, chunk_i, 0)


def kernel(x, edge_index, edge_attr, W, b):
    E = edge_index.shape[1]
    T = E + N
    per_w = -(-T // (WORKERS * CHUNK)) * CHUNK
    EP = per_w * WORKERS
    nchunk = per_w // CHUNK

    row = edge_index[0]
    col = edge_index[1]
    loop = jnp.arange(N, dtype=row.dtype)
    padv = jnp.full((EP - T,), N, row.dtype)
    rowcat = jnp.concatenate([row, loop, padv])
    colcat = jnp.concatenate([col, loop, padv])
    eacat = jnp.concatenate([edge_attr, jnp.ones((EP - E,), edge_attr.dtype)])

    wcat = jnp.concatenate([W[:C], W[C:]], axis=1)
    bcat = jnp.concatenate([b, jnp.zeros((4,), b.dtype)]).reshape(1, 8)

    tab = pl.pallas_call(
        _tc_tables_body,
        out_shape=jax.ShapeDtypeStruct((NP, 8), jnp.float32),
    )(x, wcat, bcat)

    mesh = plsc.VectorSubcoreMesh(core_axis_name="c", subcore_axis_name="s")
    sc_params = pltpu.CompilerParams(needs_layout_passes=False,
                                     use_tc_tiling_on_sc=False)

    amax_parts = pl.kernel(
        functools.partial(_sc_amax_body, per_w, nchunk),
        out_type=jax.ShapeDtypeStruct((WORKERS * NPW,), jnp.float32),
        mesh=mesh,
        compiler_params=sc_params,
        scratch_types=[
            pltpu.VMEM((NPW,), jnp.float32),
            pltpu.VMEM((CHUNK,), jnp.int32),
            pltpu.VMEM((CHUNK,), jnp.int32),
            pltpu.VMEM((CHUNK,), jnp.float32),
            pltpu.VMEM((CHUNK, 8), jnp.float32),
            pltpu.VMEM((CHUNK, 8), jnp.float32),
        ],
    )(rowcat, colcat, eacat, tab)

    amax_g = pl.kernel(
        _sc_reduce_body,
        out_type=jax.ShapeDtypeStruct((NPW,), jnp.float32),
        mesh=mesh,
        compiler_params=sc_params,
        scratch_types=[
            pltpu.VMEM((RED,), jnp.float32),
            pltpu.VMEM((RED,), jnp.float32),
        ],
    )(amax_parts)

    zeros2d = jnp.zeros((NP, 4), jnp.float32)
    exbuf, sum_parts = pl.kernel(
        functools.partial(_sc_ex_body, per_w, nchunk),
        out_type=(
            jax.ShapeDtypeStruct((EP, 4), jnp.float32),
            jax.ShapeDtypeStruct((NC, NP, 4), jnp.float32),
        ),
        mesh=mesh,
        compiler_params=sc_params,
        scratch_types=[
            pltpu.VMEM((CHUNK,), jnp.int32),
            pltpu.VMEM((CHUNK,), jnp.int32),
            pltpu.VMEM((CHUNK,), jnp.float32),
            pltpu.VMEM((CHUNK, 8), jnp.float32),
            pltpu.VMEM((CHUNK, 8), jnp.float32),
            pltpu.VMEM((CHUNK, 4), jnp.float32),
            pltpu.VMEM((CHUNK, 4), jnp.float32),
            pltpu.VMEM_SHARED((NP, 4), jnp.float32),
        ],
    )(rowcat, colcat, eacat, tab, amax_g.reshape(NP, 4), zeros2d)

    out_full = pl.kernel(
        functools.partial(_sc_norm_body, per_w, nchunk),
        out_type=jax.ShapeDtypeStruct((EP, 4), jnp.float32),
        mesh=mesh,
        compiler_params=sc_params,
        scratch_types=[
            pltpu.VMEM((CHUNK,), jnp.int32),
            pltpu.VMEM((CHUNK, 4), jnp.float32),
            pltpu.VMEM((CHUNK, 4), jnp.float32),
            pltpu.VMEM((CHUNK, 4), jnp.float32),
            pltpu.VMEM((CHUNK, 4), jnp.float32),
        ],
    )(rowcat, exbuf, sum_parts)

    alpha = out_full[:T]
    edge_index2 = jnp.stack([rowcat[:T], colcat[:T]])
    return alpha, edge_index2
